# baseline (speedup 1.0000x reference)
import functools

import jax
import jax.numpy as jnp
from jax import lax
from jax.experimental import pallas as pl
from jax.experimental.pallas import tpu as pltpu
from jax.experimental.pallas import tpu_sc as plsc

N = 50000
E = 800000
B = 128
H = 64
NC = 2
NS = 16
NP = 50176
EP = 802816
STR = NP // NS
CHUNK = 128
CL = EP // NS // CHUNK
CS = EP // (NC * NS) // CHUNK
RB = 1024
NRB = NP // RB
HH = H // NC

_mesh = plsc.VectorSubcoreMesh(
    core_axis_name="c", subcore_axis_name="s", num_cores=NC, num_subcores=NS
)


GS = 8
GSS = 8


def _sef_body(ef_hbm, dstf_hbm, z_hbm, out_hbm, acc, dbuf, rb2, rb, sem_g, sem_s, sem_i):
    c = lax.axis_index("c")
    s = lax.axis_index("s")
    w = c * NS + s
    base = w * 195 + jnp.minimum(w, 10)
    cnt = 195 + (w < 10).astype(jnp.int32)
    ngrp = (cnt + GSS - 1) // GSS
    pltpu.sync_copy(z_hbm, acc.at[pl.ds(s * STR, STR)])
    for b in range(4):
        pltpu.sync_copy(z_hbm.at[pl.ds(0, CHUNK)], rb.at[b])
    plsc.subcore_barrier()

    def stage(g, start):
        d = pltpu.make_async_copy(
            dstf_hbm.at[pl.ds(base + g * GSS, GSS)], dbuf.at[g % 2], sem_i.at[g % 2]
        )
        d.start() if start else d.wait()

    def rows(j, start):
        d = pltpu.make_async_copy(
            ef_hbm.at[:, pl.ds((base + j) * CHUNK, CHUNK)], rb2.at[j % 4],
            sem_g.at[j % 4],
        )
        d.start() if start else d.wait()

    def scat(j, start):
        args = (rb.at[j % 4], acc.at[dbuf.at[(j // GSS) % 2, j % GSS]], sem_s.at[j % 4])
        if start:
            pltpu.async_copy(*args, add=True)
        else:
            pltpu.make_async_copy(*args).wait()

    stage(0, True)
    stage(0, False)
    stage(1, True)
    rows(0, True)
    rows(1, True)
    rows(2, True)

    def body(j, carry):
        rows(j, False)

        @pl.when(jnp.logical_and((j + 3) % GSS == 0, j + 3 < cnt))
        def _():
            stage((j + 3) // GSS, False)

        @pl.when(j >= 1)
        def _():
            scat(j - 1, False)

        @pl.when(jnp.logical_and(j % GSS == 0, jnp.logical_and(j >= 1, j // GSS + 1 < ngrp)))
        def _():
            stage(j // GSS + 1, True)

        @pl.when(j + 3 < cnt)
        def _():
            rows(j + 3, True)

        buf2 = rb2.at[j % 4]
        buf = rb.at[j % 4]
        iot = lax.iota(jnp.int32, 16)
        for k in range(8):
            ridx = iot + 16 * k
            for r in range(6):
                plsc.store_scatter(
                    buf,
                    [ridx, jnp.full((16,), r, jnp.int32)],
                    buf2[r, pl.ds(16 * k, 16)],
                )

        scat(j, True)
        return carry

    lax.fori_loop(0, cnt, body, 0)
    scat(cnt - 1, False)
    plsc.subcore_barrier()
    pltpu.sync_copy(acc.at[pl.ds(s * STR, STR)], out_hbm.at[c, pl.ds(s * STR, STR)])


_sef_call = functools.partial(
    pl.kernel,
    out_type=jax.ShapeDtypeStruct((NC, NP, 16), jnp.float32),
    mesh=_mesh,
    compiler_params=pltpu.CompilerParams(use_tc_tiling_on_sc=False, needs_layout_passes=False),
    scratch_types=[
        pltpu.VMEM_SHARED((NP, 16), jnp.float32),
        pltpu.VMEM((2, GSS, CHUNK), jnp.int32),
        pltpu.VMEM((4, 8, CHUNK), jnp.float32),
        pltpu.VMEM((4, CHUNK, 16), jnp.float32),
        pltpu.SemaphoreType.DMA((4,)),
        pltpu.SemaphoreType.DMA((4,)),
        pltpu.SemaphoreType.DMA((2,)),
    ],
)(_sef_body)


def _layer_body(h_hbm, eagg_hbm, srcc_hbm, dstc_hbm, out_hbm, acc, sbuf, dbuf, rb, sem_g, sem_s, sem_i):
    c = lax.axis_index("c")
    s = lax.axis_index("s")
    base = s * 390 + jnp.minimum(s, 10)
    cnt = 390 + (s < 10).astype(jnp.int32)
    ngrp = (cnt + GS - 1) // GS
    pltpu.sync_copy(eagg_hbm.at[c, pl.ds(s * STR, STR)], acc.at[pl.ds(s * STR, STR)])
    plsc.subcore_barrier()
    h_half = h_hbm.at[c]

    def stage(g, start):
        for src_hbm, buf in ((srcc_hbm, sbuf), (dstc_hbm, dbuf)):
            d = pltpu.make_async_copy(
                src_hbm.at[pl.ds(base + g * GS, GS)], buf.at[g % 2], sem_i.at[g % 2]
            )
            d.start() if start else d.wait()

    def rows(j, start):
        d = pltpu.make_async_copy(
            h_half.at[sbuf.at[(j // GS) % 2, j % GS]], rb.at[j % 4], sem_g.at[j % 4]
        )
        d.start() if start else d.wait()

    def scat(j, start):
        args = (rb.at[j % 4], acc.at[dbuf.at[(j // GS) % 2, j % GS]], sem_s.at[j % 4])
        if start:
            pltpu.async_copy(*args, add=True)
        else:
            pltpu.make_async_copy(*args).wait()

    stage(0, True)
    stage(0, False)
    stage(1, True)
    rows(0, True)
    rows(1, True)
    rows(2, True)

    def body(j, carry):
        rows(j, False)

        @pl.when(jnp.logical_and((j + 3) % GS == 0, j + 3 < cnt))
        def _():
            stage((j + 3) // GS, False)

        @pl.when(j >= 1)
        def _():
            scat(j - 1, False)

        @pl.when(jnp.logical_and(j % GS == 0, jnp.logical_and(j >= 1, j // GS + 1 < ngrp)))
        def _():
            stage(j // GS + 1, True)

        @pl.when(j + 3 < cnt)
        def _():
            rows(j + 3, True)

        scat(j, True)
        return carry

    lax.fori_loop(0, cnt, body, 0)
    scat(cnt - 1, False)
    plsc.subcore_barrier()
    pltpu.sync_copy(acc.at[pl.ds(s * STR, STR)], out_hbm.at[c, pl.ds(s * STR, STR)])


_layer_call = functools.partial(
    pl.kernel,
    out_type=jax.ShapeDtypeStruct((NC, NP, HH), jnp.float32),
    mesh=_mesh,
    compiler_params=pltpu.CompilerParams(use_tc_tiling_on_sc=False),
    scratch_types=[
        pltpu.VMEM_SHARED((NP, HH), jnp.float32),
        pltpu.VMEM((2, GS, CHUNK), jnp.int32),
        pltpu.VMEM((2, GS, CHUNK), jnp.int32),
        pltpu.VMEM((4, CHUNK, HH), jnp.float32),
        pltpu.SemaphoreType.DMA((4,)),
        pltpu.SemaphoreType.DMA((4,)),
        pltpu.SemaphoreType.DMA((2,)),
    ],
)(_layer_body)


FB = 14
FQR = 6272 // FB
FEB = 98
FER = EP // FEB


def _fmt_body(ei_ref, src_ref, dst_ref):
    b = pl.program_id(0)
    r0 = lax.broadcasted_iota(jnp.int32, (FQR, CHUNK), 0)
    c0 = lax.broadcasted_iota(jnp.int32, (FQR, CHUNK), 1)
    q = (b * FQR + r0) * CHUNK + c0
    valid = q < E
    src_ref[...] = jnp.where(valid, ei_ref[0].reshape(FQR, CHUNK), 0)
    dst_ref[...] = jnp.where(
        valid, ei_ref[1].reshape(FQR, CHUNK), N + jnp.remainder(q, NP - N)
    )


def _fmt_call(edge_index):
    return pl.pallas_call(
        _fmt_body,
        grid=(FB,),
        in_specs=[pl.BlockSpec((2, FQR * CHUNK), lambda b: (0, b))],
        out_specs=[
            pl.BlockSpec((FQR, CHUNK), lambda b: (b, 0)),
            pl.BlockSpec((FQR, CHUNK), lambda b: (b, 0)),
        ],
        out_shape=[
            jax.ShapeDtypeStruct((EP // CHUNK, CHUNK), jnp.int32),
            jax.ShapeDtypeStruct((EP // CHUNK, CHUNK), jnp.int32),
        ],
    )(edge_index)


def _eft_body(ef_ref, out_ref):
    x = ef_ref[...]
    xp = jnp.concatenate([x, jnp.zeros((x.shape[0], 2), jnp.float32)], axis=1)
    out_ref[...] = xp.T


def _eft_call(edge_feats):
    return pl.pallas_call(
        _eft_body,
        grid=(98,),
        in_specs=[pl.BlockSpec((FER, 6), lambda b: (b, 0))],
        out_specs=pl.BlockSpec((8, FER), lambda b: (0, b)),
        out_shape=jax.ShapeDtypeStruct((8, E), jnp.float32),
    )(edge_feats)


def _embed_body(nf_ref, sef_ref, wn_ref, we_ref, h_ref, ea_ref):
    h_ref[0] = jnp.dot(nf_ref[...], wn_ref[0], preferred_element_type=jnp.float32)
    ea_ref[0] = jnp.dot(
        sef_ref[0] + sef_ref[1], we_ref[0], preferred_element_type=jnp.float32
    )


def _embed_call(nf_p, sef, wn_p, we_p):
    return pl.pallas_call(
        _embed_body,
        grid=(NC, NRB),
        in_specs=[
            pl.BlockSpec((RB, 32), lambda c, r: (r, 0)),
            pl.BlockSpec((NC, RB, 16), lambda c, r: (0, r, 0)),
            pl.BlockSpec((1, 32, HH), lambda c, r: (c, 0, 0)),
            pl.BlockSpec((1, 16, HH), lambda c, r: (c, 0, 0)),
        ],
        out_specs=[
            pl.BlockSpec((1, RB, HH), lambda c, r: (c, r, 0)),
            pl.BlockSpec((1, RB, HH), lambda c, r: (c, r, 0)),
        ],
        out_shape=[
            jax.ShapeDtypeStruct((NC, NP, HH), jnp.float32),
            jax.ShapeDtypeStruct((NC, NP, HH), jnp.float32),
        ],
    )(nf_p, sef, wn_p, we_p)


def _matmul_body(acc_ref, w_ref, h_ref):
    z = jnp.dot(
        acc_ref[0], w_ref[0, :HH], preferred_element_type=jnp.float32
    ) + jnp.dot(acc_ref[1], w_ref[0, HH:], preferred_element_type=jnp.float32)
    h_ref[0] = jnp.maximum(z, 0.0)


def _matmul_call(acc, w):
    return pl.pallas_call(
        _matmul_body,
        grid=(NC, NRB),
        in_specs=[
            pl.BlockSpec((NC, RB, HH), lambda c, r: (0, r, 0)),
            pl.BlockSpec((1, H, HH), lambda c, r: (c, 0, 0)),
        ],
        out_specs=pl.BlockSpec((1, RB, HH), lambda c, r: (c, r, 0)),
        out_shape=jax.ShapeDtypeStruct((NC, NP, HH), jnp.float32),
    )(acc, w)


def _readout_body(h_ref, ids_ref, g_ref):
    r = pl.program_id(1)
    ids = ids_ref[0, 0]
    oh = (ids[:, None] == lax.broadcasted_iota(jnp.int32, (1, B), 1)).astype(
        jnp.float32
    )
    contrib = lax.dot_general(
        oh, h_ref[0], (((0,), (0,)), ((), ())), preferred_element_type=jnp.float32
    )

    @pl.when(r == 0)
    def _():
        g_ref[0] = jnp.zeros_like(g_ref[0])

    g_ref[0] += contrib


def _readout_call(h, ids_p):
    return pl.pallas_call(
        _readout_body,
        grid=(NC, NRB),
        in_specs=[
            pl.BlockSpec((1, RB, HH), lambda c, r: (c, r, 0)),
            pl.BlockSpec((1, 1, RB), lambda c, r: (r, 0, 0)),
        ],
        out_specs=pl.BlockSpec((1, B, HH), lambda c, r: (c, 0, 0)),
        out_shape=jax.ShapeDtypeStruct((NC, B, HH), jnp.float32),
    )(h, ids_p)


def _head_body(g_ref, pe_ref, wp_ref, wo_ref, o_ref):
    p = jnp.dot(pe_ref[...], wp_ref[...], preferred_element_type=jnp.float32)
    z = (
        jnp.dot(g_ref[0], wo_ref[:HH], preferred_element_type=jnp.float32)
        + jnp.dot(g_ref[1], wo_ref[HH:H], preferred_element_type=jnp.float32)
        + jnp.dot(p, wo_ref[H:], preferred_element_type=jnp.float32)
    )
    o_ref[...] = 1.0 / (1.0 + jnp.exp(-z))


def _head_call(g, pe, wp, wo):
    return pl.pallas_call(
        _head_body,
        out_shape=jax.ShapeDtypeStruct((B, 1), jnp.float32),
    )(g, pe, wp, wo)


def kernel(node_feats, edge_feats, protein_embedding, W_node, W_edge, W_layers,
           W_prot, W_out, edge_index, node_graph_ids):
    f32 = jnp.float32
    nd = node_feats.shape[1]
    ed = edge_feats.shape[1]
    nl = W_layers.shape[0]

    nf_p = jnp.pad(node_feats.astype(f32), ((0, NP - N), (0, 32 - nd)))
    src_f, dst_f = _fmt_call(edge_index)
    ids_p = jnp.pad(node_graph_ids, (0, NP - N), constant_values=B).reshape(
        NRB, 1, RB
    )
    z16 = jnp.zeros((STR, 16), f32)
    wn_p = jnp.pad(W_node.astype(f32), ((0, 32 - nd), (0, 0)))
    wn_p = wn_p.reshape(32, NC, HH).transpose(1, 0, 2)
    we_p = jnp.pad(W_edge.astype(f32), ((0, 16 - ed), (0, 0)))
    we_p = we_p.reshape(16, NC, HH).transpose(1, 0, 2)
    wl = W_layers.astype(f32).reshape(nl, H, NC, HH).transpose(0, 2, 1, 3)

    sef = _sef_call(_eft_call(edge_feats.astype(f32)), dst_f, z16)
    h, eagg = _embed_call(nf_p, sef, wn_p, we_p)
    for i in range(nl):
        acc = _layer_call(h, eagg, src_f, dst_f)
        h = _matmul_call(acc, wl[i])
    g = _readout_call(h, ids_p)
    return _head_call(g, protein_embedding.astype(f32), W_prot.astype(f32),
                      W_out.astype(f32))


# speedup vs baseline: 1.5425x; 1.5425x over previous
import functools

import jax
import jax.numpy as jnp
from jax import lax
from jax.experimental import pallas as pl
from jax.experimental.pallas import tpu as pltpu
from jax.experimental.pallas import tpu_sc as plsc

N = 50000
E = 800000
B = 128
H = 64
NC = 2
NS = 16
NP = 50176
EP = 802816
STR = NP // NS
CHUNK = 128
CL = EP // NS // CHUNK
CS = EP // (NC * NS) // CHUNK
RB = 1024
NRB = NP // RB
HH = H // NC

_mesh = plsc.VectorSubcoreMesh(
    core_axis_name="c", subcore_axis_name="s", num_cores=NC, num_subcores=NS
)


GS = 8
GSS = 8


def _sef_body(e0, e1, e2, e3, e4, e5, dstf_hbm, z_hbm, out_hbm, acc, dbuf, ebuf, rb, sem_s, sem_i):
    c = lax.axis_index("c")
    s = lax.axis_index("s")
    w = c * NS + s
    base = w * 195 + jnp.minimum(w, 10)
    cnt = 195 + (w < 10).astype(jnp.int32)
    ngrp = (cnt + GSS - 1) // GSS
    efs = (e0, e1, e2, e3, e4, e5)
    pltpu.sync_copy(z_hbm, acc.at[pl.ds(s * STR, STR)])
    for b in range(4):
        pltpu.sync_copy(z_hbm.at[pl.ds(0, CHUNK)], rb.at[b])
    plsc.subcore_barrier()

    def stage(g, start):
        ds = []
        ds.append(pltpu.make_async_copy(
            dstf_hbm.at[pl.ds(base + g * GSS, GSS)], dbuf.at[g % 3], sem_i.at[g % 3]
        ))
        for r in range(6):
            ds.append(pltpu.make_async_copy(
                efs[r].at[pl.ds((base + g * GSS) * CHUNK, GSS * CHUNK)],
                ebuf.at[g % 3, r], sem_i.at[g % 3]
            ))
        for d in ds:
            d.start() if start else d.wait()

    def scat(j, start):
        args = (rb.at[j % 4], acc.at[dbuf.at[(j // GSS) % 3, j % GSS]], sem_s.at[j % 4])
        if start:
            pltpu.async_copy(*args, add=True)
        else:
            pltpu.make_async_copy(*args).wait()

    stage(0, True)
    stage(1, True)

    def body(j, carry):
        g = j // GSS

        @pl.when(j % GSS == 0)
        def _():
            stage(g, False)

            @pl.when(g + 2 < ngrp)
            def _():
                stage(g + 2, True)

        buf = rb.at[j % 4]
        eb = ebuf.at[g % 3]
        off = (j % GSS) * CHUNK
        iot = lax.iota(jnp.int32, 16)
        for k in range(8):
            ridx = iot + 16 * k
            for r in range(6):
                plsc.store_scatter(
                    buf,
                    [ridx, jnp.full((16,), r, jnp.int32)],
                    eb[r, pl.ds(off + 16 * k, 16)],
                )

        @pl.when(j >= 1)
        def _():
            scat(j - 1, False)

        scat(j, True)
        return carry

    lax.fori_loop(0, cnt, body, 0)
    scat(cnt - 1, False)
    plsc.subcore_barrier()
    pltpu.sync_copy(acc.at[pl.ds(s * STR, STR)], out_hbm.at[c, pl.ds(s * STR, STR)])


_sef_call = functools.partial(
    pl.kernel,
    out_type=jax.ShapeDtypeStruct((NC, NP, 16), jnp.float32),
    mesh=_mesh,
    compiler_params=pltpu.CompilerParams(use_tc_tiling_on_sc=False, needs_layout_passes=False),
    scratch_types=[
        pltpu.VMEM_SHARED((NP, 16), jnp.float32),
        pltpu.VMEM((3, GSS, CHUNK), jnp.int32),
        pltpu.VMEM((3, 6, GSS * CHUNK), jnp.float32),
        pltpu.VMEM((4, CHUNK, 16), jnp.float32),
        pltpu.SemaphoreType.DMA((4,)),
        pltpu.SemaphoreType.DMA((3,)),
    ],
)(_sef_body)


def _layer_body(h_hbm, eagg_hbm, srcc_hbm, dstc_hbm, out_hbm, acc, sbuf, dbuf, rb, sem_g, sem_s, sem_i):
    c = lax.axis_index("c")
    s = lax.axis_index("s")
    base = s * 390 + jnp.minimum(s, 10)
    cnt = 390 + (s < 10).astype(jnp.int32)
    ngrp = (cnt + GS - 1) // GS
    pltpu.sync_copy(eagg_hbm.at[c, pl.ds(s * STR, STR)], acc.at[pl.ds(s * STR, STR)])
    plsc.subcore_barrier()
    h_half = h_hbm.at[c]

    def stage(g, start):
        for src_hbm, buf in ((srcc_hbm, sbuf), (dstc_hbm, dbuf)):
            d = pltpu.make_async_copy(
                src_hbm.at[pl.ds(base + g * GS, GS)], buf.at[g % 2], sem_i.at[g % 2]
            )
            d.start() if start else d.wait()

    def rows(j, start):
        d = pltpu.make_async_copy(
            h_half.at[sbuf.at[(j // GS) % 2, j % GS]], rb.at[j % 4], sem_g.at[j % 4]
        )
        d.start() if start else d.wait()

    def scat(j, start):
        args = (rb.at[j % 4], acc.at[dbuf.at[(j // GS) % 2, j % GS]], sem_s.at[j % 4])
        if start:
            pltpu.async_copy(*args, add=True)
        else:
            pltpu.make_async_copy(*args).wait()

    stage(0, True)
    stage(0, False)
    stage(1, True)
    rows(0, True)
    rows(1, True)
    rows(2, True)

    def body(j, carry):
        rows(j, False)

        @pl.when(jnp.logical_and((j + 3) % GS == 0, j + 3 < cnt))
        def _():
            stage((j + 3) // GS, False)

        @pl.when(j >= 1)
        def _():
            scat(j - 1, False)

        @pl.when(jnp.logical_and(j % GS == 0, jnp.logical_and(j >= 1, j // GS + 1 < ngrp)))
        def _():
            stage(j // GS + 1, True)

        @pl.when(j + 3 < cnt)
        def _():
            rows(j + 3, True)

        scat(j, True)
        return carry

    lax.fori_loop(0, cnt, body, 0)
    scat(cnt - 1, False)
    plsc.subcore_barrier()
    pltpu.sync_copy(acc.at[pl.ds(s * STR, STR)], out_hbm.at[c, pl.ds(s * STR, STR)])


_layer_call = functools.partial(
    pl.kernel,
    out_type=jax.ShapeDtypeStruct((NC, NP, HH), jnp.float32),
    mesh=_mesh,
    compiler_params=pltpu.CompilerParams(use_tc_tiling_on_sc=False),
    scratch_types=[
        pltpu.VMEM_SHARED((NP, HH), jnp.float32),
        pltpu.VMEM((2, GS, CHUNK), jnp.int32),
        pltpu.VMEM((2, GS, CHUNK), jnp.int32),
        pltpu.VMEM((4, CHUNK, HH), jnp.float32),
        pltpu.SemaphoreType.DMA((4,)),
        pltpu.SemaphoreType.DMA((4,)),
        pltpu.SemaphoreType.DMA((2,)),
    ],
)(_layer_body)


FB = 14
FQR = 6272 // FB
FEB = 98
FER = EP // FEB


def _fmt_body(ei_ref, src_ref, dst_ref):
    b = pl.program_id(0)
    r0 = lax.broadcasted_iota(jnp.int32, (FQR, CHUNK), 0)
    c0 = lax.broadcasted_iota(jnp.int32, (FQR, CHUNK), 1)
    q = (b * FQR + r0) * CHUNK + c0
    valid = q < E
    src_ref[...] = jnp.where(valid, ei_ref[0].reshape(FQR, CHUNK), 0)
    dst_ref[...] = jnp.where(
        valid, ei_ref[1].reshape(FQR, CHUNK), N + jnp.remainder(q, NP - N)
    )


def _fmt_call(edge_index):
    return pl.pallas_call(
        _fmt_body,
        grid=(FB,),
        in_specs=[pl.BlockSpec((2, FQR * CHUNK), lambda b: (0, b))],
        out_specs=[
            pl.BlockSpec((FQR, CHUNK), lambda b: (b, 0)),
            pl.BlockSpec((FQR, CHUNK), lambda b: (b, 0)),
        ],
        out_shape=[
            jax.ShapeDtypeStruct((EP // CHUNK, CHUNK), jnp.int32),
            jax.ShapeDtypeStruct((EP // CHUNK, CHUNK), jnp.int32),
        ],
    )(edge_index)


def _embed_body(nf_ref, sef_ref, wn_ref, we_ref, h_ref, ea_ref):
    h_ref[0] = jnp.dot(nf_ref[...], wn_ref[0], preferred_element_type=jnp.float32)
    ea_ref[0] = jnp.dot(
        sef_ref[0] + sef_ref[1], we_ref[0], preferred_element_type=jnp.float32
    )


def _embed_call(nf_p, sef, wn_p, we_p):
    return pl.pallas_call(
        _embed_body,
        grid=(NC, NRB),
        in_specs=[
            pl.BlockSpec((RB, 32), lambda c, r: (r, 0)),
            pl.BlockSpec((NC, RB, 16), lambda c, r: (0, r, 0)),
            pl.BlockSpec((1, 32, HH), lambda c, r: (c, 0, 0)),
            pl.BlockSpec((1, 16, HH), lambda c, r: (c, 0, 0)),
        ],
        out_specs=[
            pl.BlockSpec((1, RB, HH), lambda c, r: (c, r, 0)),
            pl.BlockSpec((1, RB, HH), lambda c, r: (c, r, 0)),
        ],
        out_shape=[
            jax.ShapeDtypeStruct((NC, NP, HH), jnp.float32),
            jax.ShapeDtypeStruct((NC, NP, HH), jnp.float32),
        ],
    )(nf_p, sef, wn_p, we_p)


def _matmul_body(acc_ref, w_ref, h_ref):
    z = jnp.dot(
        acc_ref[0], w_ref[0, :HH], preferred_element_type=jnp.float32
    ) + jnp.dot(acc_ref[1], w_ref[0, HH:], preferred_element_type=jnp.float32)
    h_ref[0] = jnp.maximum(z, 0.0)


def _matmul_call(acc, w):
    return pl.pallas_call(
        _matmul_body,
        grid=(NC, NRB),
        in_specs=[
            pl.BlockSpec((NC, RB, HH), lambda c, r: (0, r, 0)),
            pl.BlockSpec((1, H, HH), lambda c, r: (c, 0, 0)),
        ],
        out_specs=pl.BlockSpec((1, RB, HH), lambda c, r: (c, r, 0)),
        out_shape=jax.ShapeDtypeStruct((NC, NP, HH), jnp.float32),
    )(acc, w)


def _readout_body(h_ref, ids_ref, g_ref):
    r = pl.program_id(1)
    ids = ids_ref[0, 0]
    oh = (ids[:, None] == lax.broadcasted_iota(jnp.int32, (1, B), 1)).astype(
        jnp.float32
    )
    contrib = lax.dot_general(
        oh, h_ref[0], (((0,), (0,)), ((), ())), preferred_element_type=jnp.float32
    )

    @pl.when(r == 0)
    def _():
        g_ref[0] = jnp.zeros_like(g_ref[0])

    g_ref[0] += contrib


def _readout_call(h, ids_p):
    return pl.pallas_call(
        _readout_body,
        grid=(NC, NRB),
        in_specs=[
            pl.BlockSpec((1, RB, HH), lambda c, r: (c, r, 0)),
            pl.BlockSpec((1, 1, RB), lambda c, r: (r, 0, 0)),
        ],
        out_specs=pl.BlockSpec((1, B, HH), lambda c, r: (c, 0, 0)),
        out_shape=jax.ShapeDtypeStruct((NC, B, HH), jnp.float32),
    )(h, ids_p)


def _head_body(g_ref, pe_ref, wp_ref, wo_ref, o_ref):
    p = jnp.dot(pe_ref[...], wp_ref[...], preferred_element_type=jnp.float32)
    z = (
        jnp.dot(g_ref[0], wo_ref[:HH], preferred_element_type=jnp.float32)
        + jnp.dot(g_ref[1], wo_ref[HH:H], preferred_element_type=jnp.float32)
        + jnp.dot(p, wo_ref[H:], preferred_element_type=jnp.float32)
    )
    o_ref[...] = 1.0 / (1.0 + jnp.exp(-z))


def _head_call(g, pe, wp, wo):
    return pl.pallas_call(
        _head_body,
        out_shape=jax.ShapeDtypeStruct((B, 1), jnp.float32),
    )(g, pe, wp, wo)


def kernel(node_feats, edge_feats, protein_embedding, W_node, W_edge, W_layers,
           W_prot, W_out, edge_index, node_graph_ids):
    f32 = jnp.float32
    nd = node_feats.shape[1]
    ed = edge_feats.shape[1]
    nl = W_layers.shape[0]

    nf_p = jnp.pad(node_feats.astype(f32), ((0, NP - N), (0, 32 - nd)))
    src_f, dst_f = _fmt_call(edge_index)
    ids_p = jnp.pad(node_graph_ids, (0, NP - N), constant_values=B).reshape(
        NRB, 1, RB
    )
    z16 = jnp.zeros((STR, 16), f32)
    wn_p = jnp.pad(W_node.astype(f32), ((0, 32 - nd), (0, 0)))
    wn_p = wn_p.reshape(32, NC, HH).transpose(1, 0, 2)
    we_p = jnp.pad(W_edge.astype(f32), ((0, 16 - ed), (0, 0)))
    we_p = we_p.reshape(16, NC, HH).transpose(1, 0, 2)
    wl = W_layers.astype(f32).reshape(nl, H, NC, HH).transpose(0, 2, 1, 3)

    ef32 = edge_feats.astype(f32)
    cols = [jnp.pad(ef32[:, ci], (0, EP - E)) for ci in range(6)]
    sef = _sef_call(*cols, dst_f, z16)
    h, eagg = _embed_call(nf_p, sef, wn_p, we_p)
    for i in range(nl):
        acc = _layer_call(h, eagg, src_f, dst_f)
        h = _matmul_call(acc, wl[i])
    g = _readout_call(h, ids_p)
    return _head_call(g, protein_embedding.astype(f32), W_prot.astype(f32),
                      W_out.astype(f32))
